# baseline (speedup 1.0000x reference)
import functools

import jax
import jax.numpy as jnp
from jax import lax
from jax.experimental import pallas as pl
from jax.experimental.pallas import tpu as pltpu
from jax.experimental.pallas import tpu_sc as plsc

_B = 4096
_N = 1024
_NC = 2
_NS = 16
_NW = _NC * _NS
_ROWS_PER_W = _B // _NW
_CHUNK = 8
_NCHUNK = _ROWS_PER_W // _CHUNK
_L = 16
_NSL = _N // _L


def _make_sc_kernel():
    mesh = plsc.VectorSubcoreMesh(core_axis_name="c", subcore_axis_name="s")

    @functools.partial(
        pl.kernel,
        mesh=mesh,
        out_type=jax.ShapeDtypeStruct((_B, _N), jnp.float32),
        scratch_types=[
            pltpu.VMEM((_CHUNK, _N), jnp.float32),
            pltpu.VMEM((_CHUNK, _N), jnp.float32),
        ],
    )
    def sc_kernel(r_hbm, out_hbm, in_v, out_v):
        wid = lax.axis_index("s") * _NC + lax.axis_index("c")
        base = wid * _ROWS_PER_W

        ones_i = jnp.ones((_L,), jnp.int32)
        zeros_i = jnp.zeros((_L,), jnp.int32)
        ones_f = jnp.ones((_L,), jnp.float32)
        zeros_f = jnp.zeros((_L,), jnp.float32)

        def chunk_body(c, carry):
            row0 = base + c * _CHUNK
            pltpu.sync_copy(r_hbm.at[pl.ds(row0, _CHUNK)], in_v)

            def row_body(i, carry2):
                def cnt_body(j, acc):
                    x = in_v[i, pl.ds(j * _L, _L)]
                    return acc + jnp.where(x < 0, ones_i, zeros_i)

                cnt = lax.fori_loop(0, _NSL, cnt_body, zeros_i)
                par = cnt & 1
                lanes = lax.iota(jnp.int32, _L)
                for s in (8, 4, 2, 1):
                    par = par ^ par.at[lanes ^ s].get(
                        mode="promise_in_bounds")
                even = par == 0

                def out_body(j, carry3):
                    x = in_v[i, pl.ds(j * _L, _L)]
                    val = jnp.where(jnp.logical_and(x < 0, even),
                                    ones_f, zeros_f)
                    out_v[i, pl.ds(j * _L, _L)] = val
                    return carry3

                return lax.fori_loop(0, _NSL, out_body, carry2)

            lax.fori_loop(0, _CHUNK, row_body, 0)
            pltpu.sync_copy(out_v, out_hbm.at[pl.ds(row0, _CHUNK)])
            return carry

        lax.fori_loop(0, _NCHUNK, chunk_body, 0)

    return sc_kernel


_SC_KERNEL = _make_sc_kernel()


def kernel(r, PermGrp):
    del PermGrp
    return _SC_KERNEL(r)
